# Initial kernel scaffold; baseline (speedup 1.0000x reference)
import jax
import jax.numpy as jnp
from jax.experimental import pallas as pl


def kernel(x, block_edge_index, pos_edge_index, neg_edge_index, W, attn_l, attn_r, bias):
    raise NotImplementedError("write your pallas kernel here")


# speedup vs baseline: 3.5068x; 3.5068x over previous
import functools

import jax
import jax.numpy as jnp
from jax import lax
from jax.experimental import pallas as pl
from jax.experimental.pallas import tpu as pltpu
from jax.experimental.pallas import tpu_sc as plsc

N = 10000
E = 320000
D = 128
NEG_SLOPE = 0.2

NC = 2
NS = 16
NW = NC * NS
L = 16

C = 80
NCH = E // NW // C
E2 = 2 * E
NCH2 = E2 // NW // C

_mesh = plsc.VectorSubcoreMesh(core_axis_name="c", subcore_axis_name="s")


def _prep_body(x_ref, w_ref, al_ref, ar_ref, h_ref, eler_ref):
    h = jnp.dot(x_ref[...], w_ref[...], preferred_element_type=jnp.float32)
    h_ref[...] = h
    el = jnp.sum(h * al_ref[...], axis=1)
    er = jnp.sum(h * ar_ref[...], axis=1)
    eler_ref[...] = jnp.concatenate([el[None, :], er[None, :]], axis=0)


def _tc_prep(x, W, al, ar):
    return pl.pallas_call(
        _prep_body,
        out_shape=[
            jax.ShapeDtypeStruct((N, D), jnp.float32),
            jax.ShapeDtypeStruct((2, N), jnp.float32),
        ],
    )(x, W, al, ar)


EH = E // NC
NCHC = EH // C
SCH = NCHC // NS
G = D // 8


@functools.partial(
    pl.kernel,
    mesh=_mesh,
    compiler_params=pltpu.CompilerParams(
        needs_layout_passes=False, use_tc_tiling_on_sc=False),
    out_type=[
        jax.ShapeDtypeStruct((NC, G * N * 8), jnp.float32),
        jax.ShapeDtypeStruct((NC, N), jnp.float32),
    ],
    scratch_types=[
        pltpu.VMEM((2 * N,), jnp.float32),
        pltpu.VMEM((C,), jnp.int32),
        pltpu.VMEM((C,), jnp.int32),
        pltpu.VMEM((C,), jnp.float32),
        pltpu.VMEM((2 * NS * C,), jnp.int32),
        pltpu.VMEM((NS * C,), jnp.float32),
        pltpu.VMEM((C,), jnp.int32),
        pltpu.VMEM((C, L), jnp.float32),
        pltpu.VMEM((N * 8,), jnp.float32),
        pltpu.VMEM((N,), jnp.float32),
        pltpu.VMEM_SHARED((4 * NS * C,), jnp.int32),
        pltpu.VMEM_SHARED((2 * NS * C,), jnp.float32),
        pltpu.SemaphoreType.DMA,
    ],
)
def _sc_gat(eler_hbm, u_hbm, v_hbm, zs_hbm, zd_hbm, hr_hbm,
            s_out, den_out,
            eler_v, ubuf, vbuf, wbuf, loc_i, loc_w, gadj,
            hbuf, s_acc, d_acc, stg_i, stg_w, semg):
    cid = lax.axis_index("c")
    sid = lax.axis_index("s")

    ar16 = lax.iota(jnp.int32, L)
    _IOTA8 = ar16 % 8
    _PAIR = ar16 // 8
    _MLO = ar16 < 8
    _MHI = ar16 >= 8
    _M0 = ar16 == 0
    _M8 = ar16 == 8

    pltpu.sync_copy(eler_hbm, eler_v)
    pltpu.sync_copy(zs_hbm, s_acc)

    @pl.when(sid == 0)
    def _():
        pltpu.sync_copy(zd_hbm, d_acc)

    def superchunk(s, carry):
        p = s % 2
        my_chunk = s * NS + sid
        pltpu.sync_copy(u_hbm.at[cid, my_chunk], ubuf)
        pltpu.sync_copy(v_hbm.at[cid, my_chunk], vbuf)
        for j in range(C // L):
            uu = ubuf[pl.ds(j * L, L)]
            vv = vbuf[pl.ds(j * L, L)]
            t = plsc.load_gather(eler_v, [uu]) + plsc.load_gather(
                eler_v, [vv + N])
            wbuf[pl.ds(j * L, L)] = jnp.exp(
                jnp.where(t >= 0.0, t, t * NEG_SLOPE))
        ib = p * (2 * NS * C)
        wb = p * (NS * C)
        pltpu.sync_copy(ubuf, stg_i.at[pl.ds(ib + sid * C, C)])
        pltpu.sync_copy(vbuf, stg_i.at[pl.ds(ib + (NS + sid) * C, C)])
        pltpu.sync_copy(wbuf, stg_w.at[pl.ds(wb + sid * C, C)])
        plsc.subcore_barrier()
        pltpu.sync_copy(stg_i.at[pl.ds(ib, 2 * NS * C)], loc_i)
        pltpu.sync_copy(stg_w.at[pl.ds(wb, NS * C)], loc_w)

        def chunk(j, carry2):
            for k in range(C // L):
                uu = loc_i[pl.ds(j * C + k * L, L)]
                gadj[pl.ds(k * L, L)] = uu + sid * N
            cpg = pltpu.async_copy(hr_hbm.at[gadj], hbuf, semg)
            cpg.wait()
            for r2 in range(C // 2):
                widx = _PAIR + (j * C + 2 * r2)
                vp = plsc.load_gather(
                    loc_i, [widx + NS * C])
                hp = plsc.load_gather(hbuf, [_PAIR + 2 * r2, _IOTA8])
                wp = plsc.load_gather(loc_w, [widx])
                val = hp * wp
                sidx = vp * 8 + _IOTA8
                plsc.addupdate_scatter(s_acc, [sidx], val, mask=_MLO)
                plsc.addupdate_scatter(s_acc, [sidx], val, mask=_MHI)

                @pl.when(sid == 0)
                def _():
                    plsc.addupdate_scatter(d_acc, [vp], wp, mask=_M0)
                    plsc.addupdate_scatter(d_acc, [vp], wp, mask=_M8)
            return carry2

        lax.fori_loop(0, NS, chunk, 0)
        return carry

    lax.fori_loop(0, SCH, superchunk, 0)

    pltpu.sync_copy(s_acc, s_out.at[cid, pl.ds(sid * N * 8, N * 8)])

    @pl.when(sid == 0)
    def _():
        pltpu.sync_copy(d_acc, den_out.at[cid])


def _combine_body(s_ref, d_ref, b_ref, out_ref):
    s = s_ref[0] + s_ref[1]
    den = d_ref[0] + d_ref[1]
    den = jnp.where(den > 0.0, den, 1.0)
    out_ref[...] = jnp.maximum(s / den[:, None] + b_ref[...], 0.0)


def _tc_combine(s_part, den_part, b):
    return pl.pallas_call(
        _combine_body,
        out_shape=jax.ShapeDtypeStruct((N, D), jnp.float32),
    )(s_part, den_part, b)


@functools.partial(
    pl.kernel,
    mesh=_mesh,
    compiler_params=pltpu.CompilerParams(needs_layout_passes=False),
    out_type=jax.ShapeDtypeStruct((NW, NCH2, C), jnp.float32),
    scratch_types=[
        pltpu.VMEM((NCH2, C), jnp.int32),
        pltpu.VMEM((NCH2, C), jnp.int32),
        pltpu.VMEM((C, D), jnp.float32),
        pltpu.VMEM((C, D), jnp.float32),
        pltpu.VMEM((L * L,), jnp.float32),
        pltpu.VMEM((NCH2, C), jnp.float32),
        pltpu.SemaphoreType.DMA,
        pltpu.SemaphoreType.DMA,
    ],
)
def _sc_score(h2_hbm, su_hbm, sv_hbm,
              sc_out,
              su_v, sv_v, rowsa, rowsb, psum, sbuf, sema, semb):
    cid = lax.axis_index("c")
    sid = lax.axis_index("s")
    wid = sid * NC + cid

    pltpu.sync_copy(su_hbm.at[wid], su_v)
    pltpu.sync_copy(sv_hbm.at[wid], sv_v)

    iota16 = lax.iota(jnp.int32, L)

    def chunk(i, carry):
        cpa = pltpu.async_copy(h2_hbm.at[su_v.at[i]], rowsa, sema)
        cpb = pltpu.async_copy(h2_hbm.at[sv_v.at[i]], rowsb, semb)
        cpa.wait()
        cpb.wait()
        for g in range(C // L):
            for e in range(L):
                r = g * L + e
                acc = rowsa[r, pl.ds(0, L)] * rowsb[r, pl.ds(0, L)]
                for k in range(1, D // L):
                    acc = acc + (rowsa[r, pl.ds(k * L, L)]
                                 * rowsb[r, pl.ds(k * L, L)])
                psum[pl.ds(e * L, L)] = acc
            tot = plsc.load_gather(psum, [iota16 * L])
            for k in range(1, L):
                tot = tot + plsc.load_gather(psum, [iota16 * L + k])
            sbuf[i, pl.ds(g * L, L)] = tot
        return carry

    lax.fori_loop(0, NCH2, chunk, 0)
    pltpu.sync_copy(sbuf, sc_out.at[wid])


def kernel(x, block_edge_index, pos_edge_index, neg_edge_index,
           W, attn_l, attn_r, bias):
    h, eler = _tc_prep(x, W, attn_l.reshape(1, D), attn_r.reshape(1, D))
    eler = eler.reshape(2 * N)

    u_blk = block_edge_index[0].reshape(NC, NCHC, C)
    v_blk = block_edge_index[1].reshape(NC, NCHC, C)
    zs = jnp.zeros((N * 8,), jnp.float32)
    zd = jnp.zeros((N,), jnp.float32)
    hr = jnp.pad(h.reshape(N, G, 8), ((0, 0), (0, 0), (0, 8)))
    hr = hr.transpose(1, 0, 2).reshape(G * N, 16)
    s_part, den_part = _sc_gat(eler, u_blk, v_blk, zs, zd, hr)

    s_part = s_part.reshape(NC, G, N, 8).transpose(0, 2, 1, 3).reshape(
        NC, N, D)
    h2 = _tc_combine(s_part, den_part, bias.reshape(1, D))

    su = jnp.concatenate(
        [pos_edge_index[0], neg_edge_index[0]]).reshape(NW, NCH2, C)
    sv = jnp.concatenate(
        [pos_edge_index[1], neg_edge_index[1]]).reshape(NW, NCH2, C)
    scores = _sc_score(h2, su, sv).reshape(E2)

    return (scores[:E, None], scores[E:, None])


# speedup vs baseline: 5.3647x; 1.5298x over previous
import functools

import jax
import jax.numpy as jnp
from jax import lax
from jax.experimental import pallas as pl
from jax.experimental.pallas import tpu as pltpu
from jax.experimental.pallas import tpu_sc as plsc

N = 10000
E = 320000
D = 128
NEG_SLOPE = 0.2

NC = 2
NS = 16
NW = NC * NS
L = 16

C = 80
NCH = E // NW // C
E2 = 2 * E
NCH2 = E2 // NW // C

_mesh = plsc.VectorSubcoreMesh(core_axis_name="c", subcore_axis_name="s")


def _prep_body(x_ref, w_ref, al_ref, ar_ref, h_ref, eler_ref):
    h = jnp.dot(x_ref[...], w_ref[...], preferred_element_type=jnp.float32)
    h_ref[...] = h
    el = jnp.sum(h * al_ref[...], axis=1)
    er = jnp.sum(h * ar_ref[...], axis=1)
    eler_ref[...] = jnp.concatenate([el[None, :], er[None, :]], axis=0)


def _tc_prep(x, W, al, ar):
    return pl.pallas_call(
        _prep_body,
        out_shape=[
            jax.ShapeDtypeStruct((N, D), jnp.float32),
            jax.ShapeDtypeStruct((2, N), jnp.float32),
        ],
    )(x, W, al, ar)


EH = E // NC
NCHC = EH // C
SCH = NCHC // NS
G = D // 8


@functools.partial(
    pl.kernel,
    mesh=_mesh,
    compiler_params=pltpu.CompilerParams(
        needs_layout_passes=False, use_tc_tiling_on_sc=False),
    out_type=[
        jax.ShapeDtypeStruct((NC, G * N * 8), jnp.float32),
        jax.ShapeDtypeStruct((NC, N), jnp.float32),
    ],
    scratch_types=[
        pltpu.VMEM((2 * N,), jnp.float32),
        pltpu.VMEM((C,), jnp.int32),
        pltpu.VMEM((C,), jnp.int32),
        pltpu.VMEM((C,), jnp.float32),
        pltpu.VMEM((2 * NS * C,), jnp.int32),
        pltpu.VMEM((NS * C,), jnp.float32),
        pltpu.VMEM((2, C), jnp.int32),
        pltpu.VMEM((2, C, L), jnp.float32),
        pltpu.VMEM((N * 8,), jnp.float32),
        pltpu.VMEM((N,), jnp.float32),
        pltpu.VMEM_SHARED((4 * NS * C,), jnp.int32),
        pltpu.VMEM_SHARED((2 * NS * C,), jnp.float32),
        pltpu.SemaphoreType.DMA,
        pltpu.SemaphoreType.DMA,
    ],
)
def _sc_gat(eler_hbm, u_hbm, v_hbm, zs_hbm, zd_hbm, hr_hbm,
            s_out, den_out,
            eler_v, ubuf, vbuf, wbuf, loc_i, loc_w, gadj,
            hbuf, s_acc, d_acc, stg_i, stg_w, sg0, sg1):
    cid = lax.axis_index("c")
    sid = lax.axis_index("s")

    ar16 = lax.iota(jnp.int32, L)
    _IOTA8 = ar16 % 8
    _PAIR = ar16 // 8
    _MLO = ar16 < 8
    _MHI = ar16 >= 8
    _M0 = ar16 == 0
    _M8 = ar16 == 8

    pltpu.sync_copy(eler_hbm, eler_v)
    pltpu.sync_copy(zs_hbm, s_acc)

    @pl.when(sid == 0)
    def _():
        pltpu.sync_copy(zd_hbm, d_acc)

    def superchunk(s, carry):
        p = s % 2
        my_chunk = s * NS + sid
        pltpu.sync_copy(u_hbm.at[cid, my_chunk], ubuf)
        pltpu.sync_copy(v_hbm.at[cid, my_chunk], vbuf)
        for j in range(C // L):
            uu = ubuf[pl.ds(j * L, L)]
            vv = vbuf[pl.ds(j * L, L)]
            t = plsc.load_gather(eler_v, [uu]) + plsc.load_gather(
                eler_v, [vv + N])
            wbuf[pl.ds(j * L, L)] = jnp.exp(
                jnp.where(t >= 0.0, t, t * NEG_SLOPE))
        ib = p * (2 * NS * C)
        wb = p * (NS * C)
        pltpu.sync_copy(ubuf, stg_i.at[pl.ds(ib + sid * C, C)])
        pltpu.sync_copy(vbuf, stg_i.at[pl.ds(ib + (NS + sid) * C, C)])
        pltpu.sync_copy(wbuf, stg_w.at[pl.ds(wb + sid * C, C)])
        plsc.subcore_barrier()
        pltpu.sync_copy(stg_i.at[pl.ds(ib, 2 * NS * C)], loc_i)
        pltpu.sync_copy(stg_w.at[pl.ds(wb, NS * C)], loc_w)

        sgs = (sg0, sg1)

        def fill_gadj(j, par):
            for k in range(C // L):
                uu = loc_i[pl.ds(j * C + k * L, L)]
                gadj[par, pl.ds(k * L, L)] = uu + sid * N

        fill_gadj(0, 0)
        pltpu.async_copy(hr_hbm.at[gadj.at[0]], hbuf.at[0], sg0)

        def chunk2(jj, carry2):
            for par in range(2):
                j = 2 * jj + par
                nxt = j + 1
                oth = 1 - par

                @pl.when(nxt < NS)
                def _():
                    fill_gadj(nxt, oth)
                    pltpu.async_copy(hr_hbm.at[gadj.at[oth]], hbuf.at[oth],
                                     sgs[oth])
                pltpu.make_async_copy(hr_hbm.at[gadj.at[par]], hbuf.at[par],
                                      sgs[par]).wait()
                for r2 in range(C // 2):
                    widx = _PAIR + (j * C + 2 * r2)
                    vp = plsc.load_gather(
                        loc_i, [widx + NS * C])
                    hp = plsc.load_gather(
                        hbuf, [jnp.full((L,), par, jnp.int32),
                               _PAIR + 2 * r2, _IOTA8])
                    wp = plsc.load_gather(loc_w, [widx])
                    val = hp * wp
                    sidx = vp * 8 + _IOTA8
                    plsc.addupdate_scatter(s_acc, [sidx], val, mask=_MLO)
                    plsc.addupdate_scatter(s_acc, [sidx], val, mask=_MHI)

                    @pl.when(sid == 0)
                    def _():
                        plsc.addupdate_scatter(d_acc, [vp], wp, mask=_M0)
                        plsc.addupdate_scatter(d_acc, [vp], wp, mask=_M8)
            return carry2

        lax.fori_loop(0, NS // 2, chunk2, 0)
        return carry

    lax.fori_loop(0, SCH, superchunk, 0)

    pltpu.sync_copy(s_acc, s_out.at[cid, pl.ds(sid * N * 8, N * 8)])

    @pl.when(sid == 0)
    def _():
        pltpu.sync_copy(d_acc, den_out.at[cid])


def _combine_body(s_ref, d_ref, b_ref, out_ref):
    s = s_ref[0] + s_ref[1]
    den = d_ref[0] + d_ref[1]
    den = jnp.where(den > 0.0, den, 1.0)
    out_ref[...] = jnp.maximum(s / den[:, None] + b_ref[...], 0.0)


def _tc_combine(s_part, den_part, b):
    return pl.pallas_call(
        _combine_body,
        out_shape=jax.ShapeDtypeStruct((N, D), jnp.float32),
    )(s_part, den_part, b)


@functools.partial(
    pl.kernel,
    mesh=_mesh,
    compiler_params=pltpu.CompilerParams(needs_layout_passes=False),
    out_type=jax.ShapeDtypeStruct((NW, NCH2, C), jnp.float32),
    scratch_types=[
        pltpu.VMEM((NCH2, C), jnp.int32),
        pltpu.VMEM((NCH2, C), jnp.int32),
        pltpu.VMEM((2, C, D), jnp.float32),
        pltpu.VMEM((2, C, D), jnp.float32),
        pltpu.VMEM((L * L,), jnp.float32),
        pltpu.VMEM((C,), jnp.float32),
        pltpu.SemaphoreType.DMA,
        pltpu.SemaphoreType.DMA,
        pltpu.SemaphoreType.DMA,
        pltpu.SemaphoreType.DMA,
    ],
)
def _sc_score(h2_hbm, su_hbm, sv_hbm,
              sc_out,
              su_v, sv_v, rowsa, rowsb, psum, sbuf, sa0, sa1, sb0, sb1):
    cid = lax.axis_index("c")
    sid = lax.axis_index("s")
    wid = sid * NC + cid

    pltpu.sync_copy(su_hbm.at[wid], su_v)
    pltpu.sync_copy(sv_hbm.at[wid], sv_v)

    iota16 = lax.iota(jnp.int32, L)
    sas = (sa0, sa1)
    sbs = (sb0, sb1)

    pltpu.async_copy(h2_hbm.at[su_v.at[0]], rowsa.at[0], sa0)
    pltpu.async_copy(h2_hbm.at[sv_v.at[0]], rowsb.at[0], sb0)

    def pair(jj, carry):
        for par in range(2):
            j = 2 * jj + par
            nxt = j + 1
            oth = 1 - par

            @pl.when(nxt < NCH2)
            def _():
                pltpu.async_copy(h2_hbm.at[su_v.at[nxt]], rowsa.at[oth],
                                 sas[oth])
                pltpu.async_copy(h2_hbm.at[sv_v.at[nxt]], rowsb.at[oth],
                                 sbs[oth])
            pltpu.make_async_copy(h2_hbm.at[su_v.at[j]], rowsa.at[par],
                                  sas[par]).wait()
            pltpu.make_async_copy(h2_hbm.at[sv_v.at[j]], rowsb.at[par],
                                  sbs[par]).wait()
            for g in range(C // L):
                for e in range(L):
                    r = g * L + e
                    acc = (rowsa[par, r, pl.ds(0, L)]
                           * rowsb[par, r, pl.ds(0, L)])
                    for k in range(1, D // L):
                        acc = acc + (rowsa[par, r, pl.ds(k * L, L)]
                                     * rowsb[par, r, pl.ds(k * L, L)])
                    psum[pl.ds(e * L, L)] = acc
                tot = plsc.load_gather(psum, [iota16 * L])
                for k in range(1, L):
                    tot = tot + plsc.load_gather(psum, [iota16 * L + k])
                sbuf[pl.ds(g * L, L)] = tot
            pltpu.sync_copy(sbuf, sc_out.at[wid, j])
        return carry

    lax.fori_loop(0, NCH2 // 2, pair, 0)


def kernel(x, block_edge_index, pos_edge_index, neg_edge_index,
           W, attn_l, attn_r, bias):
    h, eler = _tc_prep(x, W, attn_l.reshape(1, D), attn_r.reshape(1, D))
    eler = eler.reshape(2 * N)

    u_blk = block_edge_index[0].reshape(NC, NCHC, C)
    v_blk = block_edge_index[1].reshape(NC, NCHC, C)
    zs = jnp.zeros((N * 8,), jnp.float32)
    zd = jnp.zeros((N,), jnp.float32)
    hr = jnp.pad(h.reshape(N, G, 8), ((0, 0), (0, 0), (0, 8)))
    hr = hr.transpose(1, 0, 2).reshape(G * N, 16)
    s_part, den_part = _sc_gat(eler, u_blk, v_blk, zs, zd, hr)

    s_part = s_part.reshape(NC, G, N, 8).transpose(0, 2, 1, 3).reshape(
        NC, N, D)
    h2 = _tc_combine(s_part, den_part, bias.reshape(1, D))

    su = jnp.concatenate(
        [pos_edge_index[0], neg_edge_index[0]]).reshape(NW, NCH2, C)
    sv = jnp.concatenate(
        [pos_edge_index[1], neg_edge_index[1]]).reshape(NW, NCH2, C)
    scores = _sc_score(h2, su, sv).reshape(E2)

    return (scores[:E, None], scores[E:, None])


# speedup vs baseline: 5.5590x; 1.0362x over previous
import functools

import jax
import jax.numpy as jnp
from jax import lax
from jax.experimental import pallas as pl
from jax.experimental.pallas import tpu as pltpu
from jax.experimental.pallas import tpu_sc as plsc

N = 10000
E = 320000
D = 128
NEG_SLOPE = 0.2

NC = 2
NS = 16
NW = NC * NS
L = 16

C = 80
NCH = E // NW // C
E2 = 2 * E
NCH2 = E2 // NW // C

_mesh = plsc.VectorSubcoreMesh(core_axis_name="c", subcore_axis_name="s")


def _prep_body(x_ref, w_ref, al_ref, ar_ref, h_ref, eler_ref):
    h = jnp.dot(x_ref[...], w_ref[...], preferred_element_type=jnp.float32)
    h_ref[...] = h
    el = jnp.sum(h * al_ref[...], axis=1)
    er = jnp.sum(h * ar_ref[...], axis=1)
    eler_ref[...] = jnp.concatenate([el[None, :], er[None, :]], axis=0)


def _tc_prep(x, W, al, ar):
    return pl.pallas_call(
        _prep_body,
        out_shape=[
            jax.ShapeDtypeStruct((N, D), jnp.float32),
            jax.ShapeDtypeStruct((2, N), jnp.float32),
        ],
    )(x, W, al, ar)


EH = E // NC
NCHC = EH // C
SCH = NCHC // NS
G = D // 8


@functools.partial(
    pl.kernel,
    mesh=_mesh,
    compiler_params=pltpu.CompilerParams(
        needs_layout_passes=False, use_tc_tiling_on_sc=False),
    out_type=[
        jax.ShapeDtypeStruct((NC, G * N * 8), jnp.float32),
        jax.ShapeDtypeStruct((NC, N), jnp.float32),
    ],
    scratch_types=[
        pltpu.VMEM((2 * N,), jnp.float32),
        pltpu.VMEM((2, C), jnp.int32),
        pltpu.VMEM((2, C), jnp.int32),
        pltpu.VMEM((C,), jnp.float32),
        pltpu.VMEM((2 * NS * C,), jnp.int32),
        pltpu.VMEM((NS * C,), jnp.float32),
        pltpu.VMEM((2, C), jnp.int32),
        pltpu.VMEM((2, C, L), jnp.float32),
        pltpu.VMEM((N * 8,), jnp.float32),
        pltpu.VMEM((N,), jnp.float32),
        pltpu.VMEM_SHARED((4 * NS * C,), jnp.int32),
        pltpu.VMEM_SHARED((2 * NS * C,), jnp.float32),
        pltpu.SemaphoreType.DMA,
        pltpu.SemaphoreType.DMA,
        pltpu.SemaphoreType.DMA,
        pltpu.SemaphoreType.DMA,
        pltpu.SemaphoreType.DMA,
        pltpu.SemaphoreType.DMA,
    ],
)
def _sc_gat(eler_hbm, u_hbm, v_hbm, zs_hbm, zd_hbm, hr_hbm,
            s_out, den_out,
            eler_v, ubuf, vbuf, wbuf, loc_i, loc_w, gadj,
            hbuf, s_acc, d_acc, stg_i, stg_w, sg0, sg1, su0, su1, sv0, sv1):
    cid = lax.axis_index("c")
    sid = lax.axis_index("s")

    ar16 = lax.iota(jnp.int32, L)
    _IOTA8 = ar16 % 8
    _PAIR = ar16 // 8
    _MLO = ar16 < 8
    _MHI = ar16 >= 8
    _M0 = ar16 == 0
    _M8 = ar16 == 8

    pltpu.sync_copy(eler_hbm, eler_v)
    pltpu.sync_copy(zs_hbm, s_acc)

    @pl.when(sid == 0)
    def _():
        pltpu.sync_copy(zd_hbm, d_acc)

    sus = (su0, su1)
    svs = (sv0, sv1)
    pltpu.async_copy(u_hbm.at[cid, sid], ubuf.at[0], su0)
    pltpu.async_copy(v_hbm.at[cid, sid], vbuf.at[0], sv0)

    def superchunk(s, par, carry):
        p = par
        my_chunk = s * NS + sid
        pltpu.make_async_copy(u_hbm.at[cid, my_chunk], ubuf.at[par],
                              sus[par]).wait()
        pltpu.make_async_copy(v_hbm.at[cid, my_chunk], vbuf.at[par],
                              svs[par]).wait()

        @pl.when(s + 1 < SCH)
        def _():
            nc_ = (s + 1) * NS + sid
            pltpu.async_copy(u_hbm.at[cid, nc_], ubuf.at[1 - par],
                             sus[1 - par])
            pltpu.async_copy(v_hbm.at[cid, nc_], vbuf.at[1 - par],
                             svs[1 - par])
        for j in range(C // L):
            uu = ubuf[par, pl.ds(j * L, L)]
            vv = vbuf[par, pl.ds(j * L, L)]
            t = plsc.load_gather(eler_v, [uu]) + plsc.load_gather(
                eler_v, [vv + N])
            wbuf[pl.ds(j * L, L)] = jnp.exp(
                jnp.where(t >= 0.0, t, t * NEG_SLOPE))
        ib = p * (2 * NS * C)
        wb = p * (NS * C)
        pltpu.sync_copy(ubuf.at[par], stg_i.at[pl.ds(ib + sid * C, C)])
        pltpu.sync_copy(vbuf.at[par], stg_i.at[pl.ds(ib + (NS + sid) * C, C)])
        pltpu.sync_copy(wbuf, stg_w.at[pl.ds(wb + sid * C, C)])
        plsc.subcore_barrier()
        pltpu.sync_copy(stg_i.at[pl.ds(ib, 2 * NS * C)], loc_i)
        pltpu.sync_copy(stg_w.at[pl.ds(wb, NS * C)], loc_w)

        sgs = (sg0, sg1)

        def fill_gadj(j, par):
            for k in range(C // L):
                uu = loc_i[pl.ds(j * C + k * L, L)]
                gadj[par, pl.ds(k * L, L)] = uu + sid * N

        fill_gadj(0, 0)
        pltpu.async_copy(hr_hbm.at[gadj.at[0]], hbuf.at[0], sg0)

        def chunk2(jj, carry2):
            for par in range(2):
                j = 2 * jj + par
                nxt = j + 1
                oth = 1 - par

                @pl.when(nxt < NS)
                def _():
                    fill_gadj(nxt, oth)
                    pltpu.async_copy(hr_hbm.at[gadj.at[oth]], hbuf.at[oth],
                                     sgs[oth])
                pltpu.make_async_copy(hr_hbm.at[gadj.at[par]], hbuf.at[par],
                                      sgs[par]).wait()
                for r2 in range(C // 2):
                    widx = _PAIR + (j * C + 2 * r2)
                    vp = plsc.load_gather(
                        loc_i, [widx + NS * C])
                    hp = plsc.load_gather(
                        hbuf, [jnp.full((L,), par, jnp.int32),
                               _PAIR + 2 * r2, _IOTA8])
                    wp = plsc.load_gather(loc_w, [widx])
                    val = hp * wp
                    sidx = vp * 8 + _IOTA8
                    plsc.addupdate_scatter(s_acc, [sidx], val, mask=_MLO)
                    plsc.addupdate_scatter(s_acc, [sidx], val, mask=_MHI)

                    @pl.when(sid == 0)
                    def _():
                        plsc.addupdate_scatter(d_acc, [vp], wp, mask=_M0)
                        plsc.addupdate_scatter(d_acc, [vp], wp, mask=_M8)
            return carry2

        lax.fori_loop(0, NS // 2, chunk2, 0)
        return carry

    def superpair(ss, carry):
        superchunk(2 * ss, 0, carry)
        superchunk(2 * ss + 1, 1, carry)
        return carry

    lax.fori_loop(0, SCH // 2, superpair, 0)
    superchunk(SCH - 1, 0, 0)

    pltpu.sync_copy(s_acc, s_out.at[cid, pl.ds(sid * N * 8, N * 8)])

    @pl.when(sid == 0)
    def _():
        pltpu.sync_copy(d_acc, den_out.at[cid])


def _combine_body(s_ref, d_ref, b_ref, out_ref):
    s = s_ref[0] + s_ref[1]
    den = d_ref[0] + d_ref[1]
    den = jnp.where(den > 0.0, den, 1.0)
    out_ref[...] = jnp.maximum(s / den[:, None] + b_ref[...], 0.0)


def _tc_combine(s_part, den_part, b):
    return pl.pallas_call(
        _combine_body,
        out_shape=jax.ShapeDtypeStruct((N, D), jnp.float32),
    )(s_part, den_part, b)


@functools.partial(
    pl.kernel,
    mesh=_mesh,
    compiler_params=pltpu.CompilerParams(needs_layout_passes=False),
    out_type=jax.ShapeDtypeStruct((NW, NCH2, C), jnp.float32),
    scratch_types=[
        pltpu.VMEM((NCH2, C), jnp.int32),
        pltpu.VMEM((NCH2, C), jnp.int32),
        pltpu.VMEM((2, C, D), jnp.float32),
        pltpu.VMEM((2, C, D), jnp.float32),
        pltpu.VMEM((L * L,), jnp.float32),
        pltpu.VMEM((C,), jnp.float32),
        pltpu.SemaphoreType.DMA,
        pltpu.SemaphoreType.DMA,
        pltpu.SemaphoreType.DMA,
        pltpu.SemaphoreType.DMA,
    ],
)
def _sc_score(h2_hbm, su_hbm, sv_hbm,
              sc_out,
              su_v, sv_v, rowsa, rowsb, psum, sbuf, sa0, sa1, sb0, sb1):
    cid = lax.axis_index("c")
    sid = lax.axis_index("s")
    wid = sid * NC + cid

    pltpu.sync_copy(su_hbm.at[wid], su_v)
    pltpu.sync_copy(sv_hbm.at[wid], sv_v)

    iota16 = lax.iota(jnp.int32, L)
    sas = (sa0, sa1)
    sbs = (sb0, sb1)

    pltpu.async_copy(h2_hbm.at[su_v.at[0]], rowsa.at[0], sa0)
    pltpu.async_copy(h2_hbm.at[sv_v.at[0]], rowsb.at[0], sb0)

    def pair(jj, carry):
        for par in range(2):
            j = 2 * jj + par
            nxt = j + 1
            oth = 1 - par

            @pl.when(nxt < NCH2)
            def _():
                pltpu.async_copy(h2_hbm.at[su_v.at[nxt]], rowsa.at[oth],
                                 sas[oth])
                pltpu.async_copy(h2_hbm.at[sv_v.at[nxt]], rowsb.at[oth],
                                 sbs[oth])
            pltpu.make_async_copy(h2_hbm.at[su_v.at[j]], rowsa.at[par],
                                  sas[par]).wait()
            pltpu.make_async_copy(h2_hbm.at[sv_v.at[j]], rowsb.at[par],
                                  sbs[par]).wait()
            for g in range(C // L):
                for e in range(L):
                    r = g * L + e
                    acc = (rowsa[par, r, pl.ds(0, L)]
                           * rowsb[par, r, pl.ds(0, L)])
                    for k in range(1, D // L):
                        acc = acc + (rowsa[par, r, pl.ds(k * L, L)]
                                     * rowsb[par, r, pl.ds(k * L, L)])
                    psum[pl.ds(e * L, L)] = acc
                tot = plsc.load_gather(psum, [iota16 * L])
                for k in range(1, L):
                    tot = tot + plsc.load_gather(psum, [iota16 * L + k])
                sbuf[pl.ds(g * L, L)] = tot
            pltpu.sync_copy(sbuf, sc_out.at[wid, j])
        return carry

    lax.fori_loop(0, NCH2 // 2, pair, 0)


def kernel(x, block_edge_index, pos_edge_index, neg_edge_index,
           W, attn_l, attn_r, bias):
    h, eler = _tc_prep(x, W, attn_l.reshape(1, D), attn_r.reshape(1, D))
    eler = eler.reshape(2 * N)

    u_blk = block_edge_index[0].reshape(NC, NCHC, C)
    v_blk = block_edge_index[1].reshape(NC, NCHC, C)
    zs = jnp.zeros((N * 8,), jnp.float32)
    zd = jnp.zeros((N,), jnp.float32)
    hr = jnp.pad(h.reshape(N, G, 8), ((0, 0), (0, 0), (0, 8)))
    hr = hr.transpose(1, 0, 2).reshape(G * N, 16)
    s_part, den_part = _sc_gat(eler, u_blk, v_blk, zs, zd, hr)

    s_part = s_part.reshape(NC, G, N, 8).transpose(0, 2, 1, 3).reshape(
        NC, N, D)
    h2 = _tc_combine(s_part, den_part, bias.reshape(1, D))

    su = jnp.concatenate(
        [pos_edge_index[0], neg_edge_index[0]]).reshape(NW, NCH2, C)
    sv = jnp.concatenate(
        [pos_edge_index[1], neg_edge_index[1]]).reshape(NW, NCH2, C)
    scores = _sc_score(h2, su, sv).reshape(E2)

    return (scores[:E, None], scores[E:, None])
